# baseline (speedup 1.0000x reference)
import jax
import jax.numpy as jnp
from jax.experimental import pallas as pl

_B = 2048
_D_IN = 784
_H = 400
_ED = 64
_K = 512
_BLK = 2048
_NBLK = _B // _BLK


def _fwd_kernel(x_ref, w1t_ref, b1_ref, w2t_ref, b2_ref, w3t_ref, b3_ref,
                w4t_ref, b4_ref, emb_ref, embt_ref,
                out_ref, bce_ref, vq_ref):
    i = pl.program_id(0)
    x = x_ref[...]
    h1 = jnp.maximum(
        jnp.dot(x.astype(jnp.bfloat16), w1t_ref[...],
                preferred_element_type=jnp.float32) + b1_ref[...], 0.0)
    z_e = (jnp.dot(h1.astype(jnp.bfloat16), w2t_ref[...],
                   preferred_element_type=jnp.float32) + b2_ref[...])

    emb = emb_ref[...]
    embt = embt_ref[...]
    e2 = jnp.sum(embt * embt, axis=0, keepdims=True)
    score = e2 - 2.0 * jnp.dot(z_e, embt, preferred_element_type=jnp.float32)
    min_s = jnp.min(score, axis=1, keepdims=True)
    iota_k = jax.lax.broadcasted_iota(jnp.int32, (_BLK, _K), 1)
    idx = jnp.min(jnp.where(score == min_s, iota_k, _K), axis=1)
    onehot = (iota_k == idx[:, None]).astype(jnp.float32)
    z_q = jnp.dot(onehot, emb, preferred_element_type=jnp.float32)

    diff = z_e - z_q
    vq_part = jnp.sum(diff * diff)

    h3 = jnp.maximum(
        jnp.dot(z_q.astype(jnp.bfloat16), w3t_ref[...],
                preferred_element_type=jnp.float32) + b3_ref[...], 0.0)
    logits = (jnp.dot(h3.astype(jnp.bfloat16), w4t_ref[...],
                      preferred_element_type=jnp.float32) + b4_ref[...])
    p = jnp.clip(jax.nn.sigmoid(logits), 1e-12, 1.0 - 1e-12)
    out_ref[...] = p
    bce_part = jnp.sum(x * jnp.log(p) + (1.0 - x) * jnp.log(1.0 - p))

    @pl.when(i == 0)
    def _init():
        bce_ref[...] = jnp.zeros_like(bce_ref)
        vq_ref[...] = jnp.zeros_like(vq_ref)

    bce_ref[...] += bce_part.reshape(1, 1)
    vq_ref[...] += vq_part.reshape(1, 1)


def _rep(shape):
    return pl.BlockSpec(shape, lambda i: (0,) * len(shape))


def _fwd(x, w1t, b1, w2t, b2, w3t, b3, w4t, b4, emb, embt):
    return pl.pallas_call(
        _fwd_kernel,
        grid=(_NBLK,),
        in_specs=[
            pl.BlockSpec((_BLK, _D_IN), lambda i: (i, 0)),
            _rep((_D_IN, _H)), _rep((1, _H)),
            _rep((_H, _ED)), _rep((1, _ED)),
            _rep((_ED, _H)), _rep((1, _H)),
            _rep((_H, _D_IN)), _rep((1, _D_IN)),
            _rep((_K, _ED)), _rep((_ED, _K)),
        ],
        out_specs=[
            pl.BlockSpec((_BLK, _D_IN), lambda i: (i, 0)),
            pl.BlockSpec((1, 1), lambda i: (0, 0)),
            pl.BlockSpec((1, 1), lambda i: (0, 0)),
        ],
        out_shape=[
            jax.ShapeDtypeStruct((_B, _D_IN), jnp.float32),
            jax.ShapeDtypeStruct((1, 1), jnp.float32),
            jax.ShapeDtypeStruct((1, 1), jnp.float32),
        ],
    )(x, w1t, b1, w2t, b2, w3t, b3, w4t, b4, emb, embt)


def kernel(x, W1, b1, W2, b2, W3, b3, W4, b4, emb):
    bf = jnp.bfloat16
    out, bce, vq = _fwd(
        x, W1.T.astype(bf), b1.reshape(1, -1), W2.T.astype(bf),
        b2.reshape(1, -1), W3.T.astype(bf), b3.reshape(1, -1),
        W4.T.astype(bf), b4.reshape(1, -1), emb, emb.T)
    reconst_loss = -bce[0, 0] / (_B * _D_IN)
    vq_loss = vq[0, 0] / _B
    return out, reconst_loss, vq_loss, vq_loss


# speedup vs baseline: 1.2474x; 1.2474x over previous
import jax
import jax.numpy as jnp
from jax.experimental import pallas as pl

_B = 2048
_D_IN = 784
_H = 400
_ED = 64
_K = 512
_BLK = 2048
_NBLK = _B // _BLK


def _fwd_kernel(x_ref, w1t_ref, b1_ref, w2t_ref, b2_ref, w3t_ref, b3_ref,
                w4t_ref, b4_ref, emb_ref, embt_ref,
                out_ref, bce_ref, vq_ref):
    i = pl.program_id(0)
    xb = x_ref[...]
    h1 = jnp.maximum(
        jnp.dot(xb, w1t_ref[...],
                preferred_element_type=jnp.float32) + b1_ref[...], 0.0)
    z_e = (jnp.dot(h1.astype(jnp.bfloat16), w2t_ref[...],
                   preferred_element_type=jnp.float32) + b2_ref[...])

    emb = emb_ref[...]
    embt = embt_ref[...]
    e2 = jnp.sum(embt * embt, axis=0, keepdims=True)
    score = e2 - 2.0 * jnp.dot(z_e, embt, preferred_element_type=jnp.float32)
    min_s = jnp.min(score, axis=1, keepdims=True)
    iota_k = jax.lax.broadcasted_iota(jnp.int32, (_BLK, _K), 1)
    idx = jnp.min(jnp.where(score == min_s, iota_k, _K), axis=1)
    onehot = (iota_k == idx[:, None]).astype(jnp.float32)
    z_q = jnp.dot(onehot, emb, preferred_element_type=jnp.float32)

    diff = z_e - z_q
    vq_part = jnp.sum(diff * diff)

    h3 = jnp.maximum(
        jnp.dot(z_q.astype(jnp.bfloat16), w3t_ref[...],
                preferred_element_type=jnp.float32) + b3_ref[...], 0.0)
    logits = (jnp.dot(h3.astype(jnp.bfloat16), w4t_ref[...],
                      preferred_element_type=jnp.float32) + b4_ref[...])
    p = jnp.clip(jax.nn.sigmoid(logits), 1e-12, 1.0 - 1e-12)
    out_ref[...] = p.astype(jnp.bfloat16)
    x = xb.astype(jnp.float32)
    bce_part = jnp.sum(x * jnp.log(p) + (1.0 - x) * jnp.log(1.0 - p))

    @pl.when(i == 0)
    def _init():
        bce_ref[...] = jnp.zeros_like(bce_ref)
        vq_ref[...] = jnp.zeros_like(vq_ref)

    bce_ref[...] += bce_part.reshape(1, 1)
    vq_ref[...] += vq_part.reshape(1, 1)


def _rep(shape):
    return pl.BlockSpec(shape, lambda i: (0,) * len(shape))


def _fwd(x, w1t, b1, w2t, b2, w3t, b3, w4t, b4, emb, embt):
    return pl.pallas_call(
        _fwd_kernel,
        grid=(_NBLK,),
        in_specs=[
            pl.BlockSpec((_BLK, _D_IN), lambda i: (i, 0)),
            _rep((_D_IN, _H)), _rep((1, _H)),
            _rep((_H, _ED)), _rep((1, _ED)),
            _rep((_ED, _H)), _rep((1, _H)),
            _rep((_H, _D_IN)), _rep((1, _D_IN)),
            _rep((_K, _ED)), _rep((_ED, _K)),
        ],
        out_specs=[
            pl.BlockSpec((_BLK, _D_IN), lambda i: (i, 0)),
            pl.BlockSpec((1, 1), lambda i: (0, 0)),
            pl.BlockSpec((1, 1), lambda i: (0, 0)),
        ],
        out_shape=[
            jax.ShapeDtypeStruct((_B, _D_IN), jnp.bfloat16),
            jax.ShapeDtypeStruct((1, 1), jnp.float32),
            jax.ShapeDtypeStruct((1, 1), jnp.float32),
        ],
    )(x, w1t, b1, w2t, b2, w3t, b3, w4t, b4, emb, embt)


def kernel(x, W1, b1, W2, b2, W3, b3, W4, b4, emb):
    bf = jnp.bfloat16
    out, bce, vq = _fwd(
        x.astype(bf), W1.T.astype(bf), b1.reshape(1, -1), W2.T.astype(bf),
        b2.reshape(1, -1), W3.T.astype(bf), b3.reshape(1, -1),
        W4.T.astype(bf), b4.reshape(1, -1), emb, emb.T)
    reconst_loss = -bce[0, 0] / (_B * _D_IN)
    vq_loss = vq[0, 0] / _B
    return out.astype(jnp.float32), reconst_loss, vq_loss, vq_loss


# speedup vs baseline: 1.2715x; 1.0193x over previous
import jax
import jax.numpy as jnp
from jax.experimental import pallas as pl

_B = 2048
_D_IN = 784
_H = 400
_ED = 64
_K = 512
_BLK = 2048
_NBLK = _B // _BLK


def _fwd_kernel(x_ref, w1t_ref, b1_ref, w2t_ref, b2_ref, w3t_ref, b3_ref,
                w4t_ref, b4_ref, emb_ref, embt_ref,
                out_ref, bce_ref, vq_ref):
    i = pl.program_id(0)
    xb = x_ref[...]
    h1 = jnp.maximum(
        jnp.dot(xb, w1t_ref[...],
                preferred_element_type=jnp.float32) + b1_ref[...], 0.0)
    z_e = (jnp.dot(h1.astype(jnp.bfloat16), w2t_ref[...],
                   preferred_element_type=jnp.float32) + b2_ref[...])

    emb = emb_ref[...]
    embt = embt_ref[...]
    e2 = jnp.sum(embt * embt, axis=0, keepdims=True)
    score = e2 - 2.0 * jnp.dot(z_e, embt, preferred_element_type=jnp.float32)
    min_s = jnp.min(score, axis=1, keepdims=True)
    iota_k = jax.lax.broadcasted_iota(jnp.int32, (_BLK, _K), 1)
    idx = jnp.min(jnp.where(score == min_s, iota_k, _K), axis=1)
    onehot = (iota_k == idx[:, None]).astype(jnp.float32)
    z_q = jnp.dot(onehot, emb, preferred_element_type=jnp.float32)

    diff = z_e - z_q
    vq_part = jnp.sum(diff * diff)

    h3 = jnp.maximum(
        jnp.dot(z_q.astype(jnp.bfloat16), w3t_ref[...],
                preferred_element_type=jnp.float32) + b3_ref[...], 0.0)
    logits = (jnp.dot(h3.astype(jnp.bfloat16), w4t_ref[...],
                      preferred_element_type=jnp.float32) + b4_ref[...])
    t = jnp.exp(-logits)
    p = 1.0 / (1.0 + t)
    out_ref[...] = p.astype(jnp.bfloat16)
    x = xb.astype(jnp.float32)
    bce_part = jnp.sum((x - 1.0) * logits - jnp.log(1.0 + t))

    @pl.when(i == 0)
    def _init():
        bce_ref[...] = jnp.zeros_like(bce_ref)
        vq_ref[...] = jnp.zeros_like(vq_ref)

    bce_ref[...] += bce_part.reshape(1, 1)
    vq_ref[...] += vq_part.reshape(1, 1)


def _rep(shape):
    return pl.BlockSpec(shape, lambda i: (0,) * len(shape))


def _fwd(x, w1t, b1, w2t, b2, w3t, b3, w4t, b4, emb, embt):
    return pl.pallas_call(
        _fwd_kernel,
        grid=(_NBLK,),
        in_specs=[
            pl.BlockSpec((_BLK, _D_IN), lambda i: (i, 0)),
            _rep((_D_IN, _H)), _rep((1, _H)),
            _rep((_H, _ED)), _rep((1, _ED)),
            _rep((_ED, _H)), _rep((1, _H)),
            _rep((_H, _D_IN)), _rep((1, _D_IN)),
            _rep((_K, _ED)), _rep((_ED, _K)),
        ],
        out_specs=[
            pl.BlockSpec((_BLK, _D_IN), lambda i: (i, 0)),
            pl.BlockSpec((1, 1), lambda i: (0, 0)),
            pl.BlockSpec((1, 1), lambda i: (0, 0)),
        ],
        out_shape=[
            jax.ShapeDtypeStruct((_B, _D_IN), jnp.bfloat16),
            jax.ShapeDtypeStruct((1, 1), jnp.float32),
            jax.ShapeDtypeStruct((1, 1), jnp.float32),
        ],
    )(x, w1t, b1, w2t, b2, w3t, b3, w4t, b4, emb, embt)


def kernel(x, W1, b1, W2, b2, W3, b3, W4, b4, emb):
    bf = jnp.bfloat16
    out, bce, vq = _fwd(
        x.astype(bf), W1.T.astype(bf), b1.reshape(1, -1), W2.T.astype(bf),
        b2.reshape(1, -1), W3.T.astype(bf), b3.reshape(1, -1),
        W4.T.astype(bf), b4.reshape(1, -1), emb, emb.T)
    reconst_loss = -bce[0, 0] / (_B * _D_IN)
    vq_loss = vq[0, 0] / _B
    return out.astype(jnp.float32), reconst_loss, vq_loss, vq_loss
